# Initial kernel scaffold; baseline (speedup 1.0000x reference)
import jax
import jax.numpy as jnp
from jax.experimental import pallas as pl


def kernel(x, edge_index, W_l1, b_l1, W_r1, g1, be1, m1, v1, W_l2, b_l2, W_r2, g2, be2, m2, v2, W_l3, b_l3, W_r3):
    raise NotImplementedError("write your pallas kernel here")


# speedup vs baseline: 3.5484x; 3.5484x over previous
import jax
import jax.numpy as jnp
from jax import lax
from jax.experimental import pallas as pl
from jax.experimental.pallas import tpu as pltpu
from jax.experimental.pallas import tpu_sc as plsc

N = 10000
D = 128
NC = 2
NS = 16
L = 16
NW = NC * NS
CHUNK = 128
N_PAD = 10112
STRIPE = N_PAD // NS


def _make_agg(E_pad: int):
    EPT = E_pad // NW
    CPT = EPT // CHUNK
    mesh = plsc.VectorSubcoreMesh(core_axis_name="c", subcore_axis_name="s",
                                  num_cores=NC, num_subcores=NS)

    out_type = jax.ShapeDtypeStruct((NC, N_PAD, D), jnp.float32)
    scratch = [
        pltpu.VMEM((CHUNK,), jnp.int32),
        pltpu.VMEM((CHUNK,), jnp.int32),
        pltpu.VMEM((CHUNK, D), jnp.float32),
        pltpu.VMEM_SHARED((N_PAD, D), jnp.float32),
        pltpu.SemaphoreType.DMA,
    ]

    def body(x_hbm, src_hbm, dst_hbm, acc_out, src_v, dst_v, rows_v,
             acc_sh, sem):
        cid = lax.axis_index("c")
        sid = lax.axis_index("s")
        wid = sid * NC + cid

        zero = jnp.zeros((L,), jnp.float32)

        def zrow(r, _):
            for cc in range(D // L):
                rows_v[r, pl.ds(cc * L, L)] = zero
            return 0
        lax.fori_loop(0, CHUNK, zrow, 0)
        off = 0
        while off < STRIPE:
            w = min(CHUNK, STRIPE - off)
            pltpu.sync_copy(rows_v.at[pl.ds(0, w)],
                            acc_sh.at[pl.ds(sid * STRIPE + off, w)])
            off += w

        plsc.subcore_barrier()

        base0 = wid * EPT

        def step(c, _):
            base = base0 + c * CHUNK
            pltpu.sync_copy(src_hbm.at[pl.ds(base, CHUNK)], src_v)
            pltpu.sync_copy(dst_hbm.at[pl.ds(base, CHUNK)], dst_v)
            pltpu.async_copy(x_hbm.at[src_v], rows_v, sem).wait()
            pltpu.sync_copy(rows_v, acc_sh.at[dst_v], add=True)
            return 0
        lax.fori_loop(0, CPT, step, 0)

        plsc.subcore_barrier()

        pltpu.sync_copy(acc_sh.at[pl.ds(sid * STRIPE, STRIPE)],
                        acc_out.at[cid, pl.ds(sid * STRIPE, STRIPE)])

    return pl.kernel(body, out_type=out_type, mesh=mesh,
                     scratch_types=scratch)


def _make_deg(E_pad: int):
    EPT = E_pad // NW
    CPT = EPT // CHUNK
    mesh = plsc.VectorSubcoreMesh(core_axis_name="c", subcore_axis_name="s",
                                  num_cores=NC, num_subcores=NS)

    out_type = jax.ShapeDtypeStruct((NC, N_PAD, D), jnp.float32)
    scratch = [
        pltpu.VMEM((CHUNK,), jnp.int32),
        pltpu.VMEM((CHUNK, D), jnp.float32),
        pltpu.VMEM((CHUNK, D), jnp.float32),
        pltpu.VMEM_SHARED((N_PAD, D), jnp.float32),
    ]

    def body(dst_hbm, deg_out, dst_v, ones_v, dchunk_v, deg_sh):
        cid = lax.axis_index("c")
        sid = lax.axis_index("s")
        wid = sid * NC + cid

        zero = jnp.zeros((L,), jnp.float32)
        one = jnp.full((L,), 1.0, jnp.float32)

        def orow(r, _):
            for cc in range(D // L):
                ones_v[r, pl.ds(cc * L, L)] = one
                dchunk_v[r, pl.ds(cc * L, L)] = zero
            return 0
        lax.fori_loop(0, CHUNK, orow, 0)
        off = 0
        while off < STRIPE:
            w = min(CHUNK, STRIPE - off)
            pltpu.sync_copy(dchunk_v.at[pl.ds(0, w)],
                            deg_sh.at[pl.ds(sid * STRIPE + off, w)])
            off += w

        plsc.subcore_barrier()

        base0 = wid * EPT

        def step(c, _):
            base = base0 + c * CHUNK
            pltpu.sync_copy(dst_hbm.at[pl.ds(base, CHUNK)], dst_v)
            pltpu.sync_copy(ones_v, deg_sh.at[dst_v], add=True)
            return 0
        lax.fori_loop(0, CPT, step, 0)

        plsc.subcore_barrier()

        pltpu.sync_copy(deg_sh.at[pl.ds(sid * STRIPE, STRIPE)],
                        deg_out.at[cid, pl.ds(sid * STRIPE, STRIPE)])

    return pl.kernel(body, out_type=out_type, mesh=mesh,
                     scratch_types=scratch)


def _dense(acc, d0, d1, x, A, B, c, do_relu: bool):
    BLK = 1000

    def body(a0_r, a1_r, d0_r, d1_r, x_r, A_r, B_r, c_r, o_r):
        s = a0_r[...] + a1_r[...]
        dg = d0_r[...] + d1_r[...]
        r = 1.0 / jnp.maximum(dg[:, 0:1], 1.0)
        h = (jnp.dot(s * r, A_r[...], preferred_element_type=jnp.float32)
             + jnp.dot(x_r[...], B_r[...], preferred_element_type=jnp.float32)
             + c_r[...])
        if do_relu:
            h = jnp.maximum(h, 0.0)
        o_r[...] = h

    row = lambda i: (i, 0)
    full = lambda i: (0, 0)
    return pl.pallas_call(
        body,
        grid=(N // BLK,),
        in_specs=[
            pl.BlockSpec((BLK, D), row),
            pl.BlockSpec((BLK, D), row),
            pl.BlockSpec((BLK, D), row),
            pl.BlockSpec((BLK, D), row),
            pl.BlockSpec((BLK, D), row),
            pl.BlockSpec((D, D), full),
            pl.BlockSpec((D, D), full),
            pl.BlockSpec((1, D), full),
        ],
        out_specs=pl.BlockSpec((BLK, D), row),
        out_shape=jax.ShapeDtypeStruct((N, D), jnp.float32),
    )(acc[0], acc[1], d0, d1, x, A, B, c)


def kernel(x, edge_index, W_l1, b_l1, W_r1, g1, be1, m1, v1,
           W_l2, b_l2, W_r2, g2, be2, m2, v2, W_l3, b_l3, W_r3):
    E = edge_index.shape[1]
    E_pad = -(-E // (NW * CHUNK)) * (NW * CHUNK)
    pad = E_pad - E
    src = jnp.concatenate([edge_index[0], jnp.zeros((pad,), jnp.int32)])
    dst = jnp.concatenate(
        [edge_index[1], jnp.full((pad,), N_PAD - 1, jnp.int32)])

    agg = _make_agg(E_pad)
    degf = _make_deg(E_pad)

    eps = 1e-5
    s1 = g1 * lax.rsqrt(v1 + eps)
    A1 = W_l1.T * s1[None, :]
    B1 = W_r1.T * s1[None, :]
    c1 = (b_l1 * s1 + be1 - m1 * s1)[None, :]
    s2 = g2 * lax.rsqrt(v2 + eps)
    A2 = W_l2.T * s2[None, :]
    B2 = W_r2.T * s2[None, :]
    c2 = (b_l2 * s2 + be2 - m2 * s2)[None, :]
    A3 = W_l3.T
    B3 = W_r3.T
    c3 = b_l3[None, :]

    deg = degf(dst)
    d0, d1 = deg[0], deg[1]
    acc1 = agg(x, src, dst)
    h1 = _dense(acc1, d0, d1, x, A1, B1, c1, True)
    acc2 = agg(h1, src, dst)
    h2 = _dense(acc2, d0, d1, h1, A2, B2, c2, True)
    acc3 = agg(h2, src, dst)
    return _dense(acc3, d0, d1, h2, A3, B3, c3, False)
